# baseline (speedup 1.0000x reference)
import functools

import jax
import jax.numpy as jnp
import numpy as np
from jax import lax
from jax.experimental import pallas as pl
from jax.experimental.pallas import tpu as pltpu
from jax.experimental.pallas import tpu_sc as plsc

_A = 100000
_TILE = 2048

_NW = 32
_P = 384
_A_SC = _NW * _P

_KS0 = np.uint32(0)
_KS1 = np.uint32(42)
_KS2 = np.uint32(_KS0 ^ _KS1 ^ np.uint32(0x1BD11BDA))
_ROT_A = (13, 15, 26, 6)
_ROT_B = (17, 29, 16, 24)
_TINY = np.float32(np.finfo(np.float32).tiny)
_LN2 = np.float32(0.6931471805599453)
_MB_SQRT2 = np.int32(0x3504F3)


def _rotl(v, r):
    return jax.lax.shift_left(v, np.uint32(r)) | jax.lax.shift_right_logical(
        v, np.uint32(32 - r)
    )


def _gumbel_bits(idx_u32):
    ks = (_KS0, _KS1, _KS2)
    x0 = jnp.zeros_like(idx_u32) + _KS0
    x1 = idx_u32 + _KS1
    rots = (_ROT_A, _ROT_B)
    for i in range(5):
        for r in rots[i % 2]:
            x0 = x0 + x1
            x1 = _rotl(x1, r)
            x1 = x1 ^ x0
        x0 = x0 + ks[(i + 1) % 3]
        x1 = x1 + ks[(i + 2) % 3] + np.uint32(i + 1)
    return x0 ^ x1


def _bits_to_uniform(bits):
    mant = jax.lax.shift_right_logical(bits, np.uint32(9)) | np.uint32(0x3F800000)
    f = jax.lax.bitcast_convert_type(mant, jnp.float32) - np.float32(1.0)
    return jnp.maximum(_TINY, f + _TINY)


def _gumbel(idx_u32):
    u = _bits_to_uniform(_gumbel_bits(idx_u32))
    return -jnp.log(-jnp.log(u))


def _poly_log(x):
    bits = jax.lax.bitcast_convert_type(x, jnp.int32)
    mb = jax.lax.bitwise_and(bits, jnp.int32(0x7FFFFF))
    e = jax.lax.shift_right_logical(bits, jnp.int32(23)) - 127
    big = mb >= _MB_SQRT2
    expo_bits = jnp.where(big, jnp.int32(0x3F000000), jnp.int32(0x3F800000))
    m = jax.lax.bitcast_convert_type(jax.lax.bitwise_or(mb, expo_bits), jnp.float32)
    ef = jnp.where(big, e + 1, e).astype(jnp.float32)
    s = (m - np.float32(1.0)) / (m + np.float32(1.0))
    z = s * s
    p = np.float32(2.0 / 9.0)
    p = p * z + np.float32(2.0 / 7.0)
    p = p * z + np.float32(2.0 / 5.0)
    p = p * z + np.float32(2.0 / 3.0)
    p = p * z + np.float32(2.0)
    return ef * _LN2 + s * p


def _gumbel_sc(idx_u32):
    u = _bits_to_uniform(_gumbel_bits(idx_u32))
    return -_poly_log(-_poly_log(u))


def _sc_logits_kernel(x_ref, w_ref, b_ref, out_ref):
    logits = jax.lax.dot_general(
        x_ref[...],
        w_ref[...],
        (((1,), (1,)), ((), ())),
        preferred_element_type=jnp.float32,
    )
    out_ref[0] = logits + b_ref[...]


def _sc_body(lg_hbm, val_hbm, idx_hbm, lg_v, val_v, idx_v):
    c = lax.axis_index("c")
    s = lax.axis_index("s")
    wid = c * 16 + s
    pltpu.sync_copy(lg_hbm.at[wid], lg_v)
    lane = lax.iota(jnp.int32, 16)

    def row(r, carry):
        base = wid * _P + r * _A
        rm = jnp.full((16,), -jnp.inf, jnp.float32)
        ri = jnp.zeros((16,), jnp.int32)
        for j in range(_P // 16):
            lv = lg_v[r, pl.ds(j * 16, 16)]
            col = lane + (wid * _P + j * 16)
            flat = (base + j * 16 + lane).astype(jnp.uint32)
            cand = lv + _gumbel_sc(flat)
            upd = cand > rm
            rm = jnp.where(upd, cand, rm)
            ri = jnp.where(upd, col, ri)
        val_v[pl.ds(r * 16, 16)] = rm
        idx_v[pl.ds(r * 16, 16)] = ri
        return carry

    lax.fori_loop(0, 128, row, 0)
    pltpu.sync_copy(val_v, val_hbm.at[wid])
    pltpu.sync_copy(idx_v, idx_hbm.at[wid])


def _policy_kernel(x_ref, w_ref, b_ref, out_v_ref, out_i_ref, best_v, best_i, *, num_blocks):
    blk = pl.program_id(0)
    B = x_ref.shape[0]
    T = w_ref.shape[0]

    @pl.when(blk == 0)
    def _init():
        best_v[...] = jnp.full((B, 1), -jnp.inf, jnp.float32)
        best_i[...] = jnp.zeros((B, 1), jnp.int32)

    logits = jax.lax.dot_general(
        x_ref[...],
        w_ref[...],
        (((1,), (1,)), ((), ())),
        preferred_element_type=jnp.float32,
    )
    logits = logits + b_ref[...]

    col = jax.lax.broadcasted_iota(jnp.int32, (B, T), 1) + (_A_SC + blk * T)
    row = jax.lax.broadcasted_iota(jnp.int32, (B, T), 0)
    flat = (row * _A + col).astype(jnp.uint32)
    cand = logits + _gumbel(flat)
    cand = jnp.where(col < _A, cand, -jnp.inf)

    m = jnp.max(cand, axis=1, keepdims=True)
    idx = jnp.min(
        jnp.where(cand == m, col, jnp.int32(0x7FFFFFFF)), axis=1, keepdims=True
    )
    better = m > best_v[...]
    best_v[...] = jnp.where(better, m, best_v[...])
    best_i[...] = jnp.where(better, idx, best_i[...])

    @pl.when(blk == num_blocks - 1)
    def _write():
        out_v_ref[...] = best_v[...]
        out_i_ref[...] = best_i[...]


def _merge_kernel(sv_ref, si_ref, tv_ref, ti_ref, out_ref, acc_v, acc_i):
    t = pl.program_id(0)
    v = sv_ref[0]
    i = si_ref[0]

    @pl.when(t == 0)
    def _init():
        acc_v[...] = v
        acc_i[...] = i

    @pl.when(t > 0)
    def _fold():
        upd = v > acc_v[...]
        acc_v[...] = jnp.where(upd, v, acc_v[...])
        acc_i[...] = jnp.where(upd, i, acc_i[...])

    @pl.when(t == _NW - 1)
    def _final():
        av = acc_v[...]
        m = jnp.max(av, axis=1, keepdims=True)
        isel = jnp.min(
            jnp.where(av == m, acc_i[...], jnp.int32(0x7FFFFFFF)),
            axis=1,
            keepdims=True,
        )
        sc_wins = m >= tv_ref[...]
        out_ref[...] = jnp.where(sc_wins, isel, ti_ref[...])


def kernel(x, W, b):
    B, D = x.shape
    A = W.shape[0]
    b2 = b.reshape(1, A)

    lg_sc = pl.pallas_call(
        _sc_logits_kernel,
        grid=(_NW,),
        in_specs=[
            pl.BlockSpec((B, D), lambda t: (0, 0)),
            pl.BlockSpec((_P, D), lambda t: (t, 0)),
            pl.BlockSpec((1, _P), lambda t: (0, t)),
        ],
        out_specs=pl.BlockSpec((1, B, _P), lambda t: (t, 0, 0)),
        out_shape=jax.ShapeDtypeStruct((_NW, B, _P), jnp.float32),
    )(x, W, b2)

    sc_call = functools.partial(
        pl.kernel,
        mesh=plsc.VectorSubcoreMesh(core_axis_name="c", subcore_axis_name="s"),
        out_type=[
            jax.ShapeDtypeStruct((_NW, B * 16), jnp.float32),
            jax.ShapeDtypeStruct((_NW, B * 16), jnp.int32),
        ],
        scratch_types=[
            pltpu.VMEM((B, _P), jnp.float32),
            pltpu.VMEM((B * 16,), jnp.float32),
            pltpu.VMEM((B * 16,), jnp.int32),
        ],
    )(_sc_body)
    sc_val, sc_idx = sc_call(lg_sc)
    sc_val = sc_val.reshape(_NW, B, 16)
    sc_idx = sc_idx.reshape(_NW, B, 16)

    G = pl.cdiv(A - _A_SC, _TILE)
    tc_val, tc_idx = pl.pallas_call(
        functools.partial(_policy_kernel, num_blocks=G),
        grid=(G,),
        in_specs=[
            pl.BlockSpec((B, D), lambda i: (0, 0)),
            pl.BlockSpec((_TILE, D), lambda i: (_A_SC // _TILE + i, 0)),
            pl.BlockSpec((1, _TILE), lambda i: (0, _A_SC // _TILE + i)),
        ],
        out_specs=[
            pl.BlockSpec((B, 1), lambda i: (0, 0)),
            pl.BlockSpec((B, 1), lambda i: (0, 0)),
        ],
        out_shape=[
            jax.ShapeDtypeStruct((B, 1), jnp.float32),
            jax.ShapeDtypeStruct((B, 1), jnp.int32),
        ],
        scratch_shapes=[
            pltpu.VMEM((B, 1), jnp.float32),
            pltpu.VMEM((B, 1), jnp.int32),
        ],
    )(x, W, b2)

    sample = pl.pallas_call(
        _merge_kernel,
        grid=(_NW,),
        in_specs=[
            pl.BlockSpec((1, B, 16), lambda t: (t, 0, 0)),
            pl.BlockSpec((1, B, 16), lambda t: (t, 0, 0)),
            pl.BlockSpec((B, 1), lambda t: (0, 0)),
            pl.BlockSpec((B, 1), lambda t: (0, 0)),
        ],
        out_specs=pl.BlockSpec((B, 1), lambda t: (0, 0)),
        out_shape=jax.ShapeDtypeStruct((B, 1), jnp.int32),
        scratch_shapes=[
            pltpu.VMEM((B, 16), jnp.float32),
            pltpu.VMEM((B, 16), jnp.int32),
        ],
    )(sc_val, sc_idx, tc_val, tc_idx)

    return sample.astype(jnp.int64)


# speedup vs baseline: 1.4236x; 1.4236x over previous
import functools

import jax
import jax.numpy as jnp
import numpy as np
from jax import lax
from jax.experimental import pallas as pl
from jax.experimental.pallas import tpu as pltpu
from jax.experimental.pallas import tpu_sc as plsc

_A = 100000
_TILE = 2048

_NW = 32
_P = 512
_A_SC = _NW * _P
_AGRP = 4
_CH = 64

_KS0 = np.uint32(0)
_KS1 = np.uint32(42)
_KS2 = np.uint32(_KS0 ^ _KS1 ^ np.uint32(0x1BD11BDA))
_ROT_A = (13, 15, 26, 6)
_ROT_B = (17, 29, 16, 24)
_TINY = np.float32(np.finfo(np.float32).tiny)
_LN2 = np.float32(0.6931471805599453)
_MB_SQRT2 = np.int32(0x3504F3)


def _rotl(v, r):
    return jax.lax.shift_left(v, np.uint32(r)) | jax.lax.shift_right_logical(
        v, np.uint32(32 - r)
    )


def _gumbel_bits(idx_u32):
    v = idx_u32 + _KS1
    x0 = v
    x1 = _rotl(v, 13) ^ v
    for r in (15, 26, 6):
        x0 = x0 + x1
        x1 = _rotl(x1, r)
        x1 = x1 ^ x0
    x0 = x0 + _KS1
    x1 = x1 + np.uint32(_KS2 + np.uint32(1))
    for r in _ROT_B:
        x0 = x0 + x1
        x1 = _rotl(x1, r)
        x1 = x1 ^ x0
    x0 = x0 + _KS2
    x1 = x1 + np.uint32(2)
    for r in _ROT_A:
        x0 = x0 + x1
        x1 = _rotl(x1, r)
        x1 = x1 ^ x0
    x1 = x1 + np.uint32(_KS1 + np.uint32(3))
    for r in _ROT_B:
        x0 = x0 + x1
        x1 = _rotl(x1, r)
        x1 = x1 ^ x0
    x0 = x0 + _KS1
    x1 = x1 + np.uint32(_KS2 + np.uint32(4))
    for r in _ROT_A:
        x0 = x0 + x1
        x1 = _rotl(x1, r)
        x1 = x1 ^ x0
    x0 = x0 + _KS2
    x1 = x1 + np.uint32(5)
    return x0 ^ x1


def _bits_to_uniform(bits):
    mant = jax.lax.shift_right_logical(bits, np.uint32(9)) | np.uint32(0x3F800000)
    f = jax.lax.bitcast_convert_type(mant, jnp.float32) - np.float32(1.0)
    return f + _TINY


def _gumbel(idx_u32):
    u = _bits_to_uniform(_gumbel_bits(idx_u32))
    return -jnp.log(-jnp.log(u))


def _neg_poly_log(x):
    bits = jax.lax.bitcast_convert_type(x, jnp.int32)
    mb = jax.lax.bitwise_and(bits, jnp.int32(0x7FFFFF))
    sh = jax.lax.shift_right_logical(bits, jnp.int32(23))
    big = mb >= _MB_SQRT2
    expo_bits = jnp.where(big, jnp.int32(0x3F000000), jnp.int32(0x3F800000))
    m = jax.lax.bitcast_convert_type(jax.lax.bitwise_or(mb, expo_bits), jnp.float32)
    ef = jnp.where(big, 126 - sh, 127 - sh).astype(jnp.float32)
    s = (m - np.float32(1.0)) / (m + np.float32(1.0))
    z = s * s
    p = np.float32(-2.0 / 7.0)
    p = p * z + np.float32(-2.0 / 5.0)
    p = p * z + np.float32(-2.0 / 3.0)
    p = p * z + np.float32(-2.0)
    return ef * _LN2 + s * p


def _gumbel_sc(idx_u32):
    u = _bits_to_uniform(_gumbel_bits(idx_u32))
    return _neg_poly_log(_neg_poly_log(u))


def _sc_logits_kernel(x_ref, w_ref, b_ref, out_ref):
    logits = jax.lax.dot_general(
        x_ref[...],
        w_ref[...],
        (((1,), (1,)), ((), ())),
        preferred_element_type=jnp.float32,
    )
    logits = logits + b_ref[...]
    for k in range(_AGRP):
        out_ref[k] = logits[:, k * _P : (k + 1) * _P]


def _sc_body(lg_hbm, val_hbm, idx_hbm, lg_v, val_v, idx_v):
    c = lax.axis_index("c")
    s = lax.axis_index("s")
    wid = c * 16 + s
    pltpu.sync_copy(lg_hbm.at[wid], lg_v)
    lane = lax.iota(jnp.int32, 16)

    def _fold(pairs, rm, ri):
        while len(pairs) > 1:
            nxt = []
            for k in range(0, len(pairs), 2):
                (va, ia), (vb, ib) = pairs[k], pairs[k + 1]
                upd = vb > va
                nxt.append((jnp.where(upd, vb, va), jnp.where(upd, ib, ia)))
            pairs = nxt
        vc, ic = pairs[0]
        upd = vc > rm
        return jnp.where(upd, vc, rm), jnp.where(upd, ic, ri)

    _NR = 4

    def rowgrp(rp, carry):
        rows = [rp * _NR + q for q in range(_NR)]

        def chunk(h, st):
            rms = list(st[:_NR])
            ris = list(st[_NR:])
            pairs = [[] for _ in range(_NR)]
            for j8 in range(_CH // 16):
                off = h * _CH + j8 * 16
                col = lane + (wid * _P + off)
                for q, r in enumerate(rows):
                    lv = lg_v[r, pl.ds(off, 16)]
                    pairs[q].append(
                        (lv + _gumbel_sc((col + r * _A).astype(jnp.uint32)), col)
                    )
            for q in range(_NR):
                rms[q], ris[q] = _fold(pairs[q], rms[q], ris[q])
            return tuple(rms) + tuple(ris)

        ninf = jnp.full((16,), -jnp.inf, jnp.float32)
        zero = jnp.zeros((16,), jnp.int32)
        st = lax.fori_loop(
            0, _P // _CH, chunk, (ninf,) * _NR + (zero,) * _NR
        )
        for q, r in enumerate(rows):
            val_v[r] = st[q]
            idx_v[r] = st[_NR + q]
        return carry

    lax.fori_loop(0, 128 // _NR, rowgrp, 0)
    pltpu.sync_copy(val_v, val_hbm.at[wid])
    pltpu.sync_copy(idx_v, idx_hbm.at[wid])


def _policy_kernel(
    x_ref, w_ref, b_ref, out_v_ref, out_i_ref, vacc, iacc, rowbase, *, num_blocks
):
    blk = pl.program_id(0)
    B = x_ref.shape[0]
    T = w_ref.shape[0]

    @pl.when(blk == 0)
    def _init():
        vacc[...] = jnp.full((B, 1), -jnp.inf, jnp.float32)
        iacc[...] = jnp.zeros((B, 1), jnp.int32)
        rowbase[...] = jax.lax.broadcasted_iota(jnp.int32, (B, T), 0) * _A

    logits = jax.lax.dot_general(
        x_ref[...],
        w_ref[...],
        (((1,), (1,)), ((), ())),
        preferred_element_type=jnp.float32,
    )
    logits = logits + b_ref[...]

    col = jax.lax.broadcasted_iota(jnp.int32, (B, T), 1) + (_A_SC + blk * T)
    flat = (rowbase[...] + col).astype(jnp.uint32)
    cand = logits + _gumbel(flat)

    cand = jnp.where(col < _A, cand, -jnp.inf)

    m = jnp.max(cand, axis=1, keepdims=True)
    idx = jnp.min(
        jnp.where(cand == m, col, jnp.int32(0x7FFFFFFF)), axis=1, keepdims=True
    )
    better = m > vacc[...]
    vacc[...] = jnp.where(better, m, vacc[...])
    iacc[...] = jnp.where(better, idx, iacc[...])

    @pl.when(blk == num_blocks - 1)
    def _write():
        out_v_ref[...] = vacc[...]
        out_i_ref[...] = iacc[...]


def _merge_kernel(sv_ref, si_ref, tv_ref, ti_ref, out_ref):
    sv = sv_ref[...]
    si = si_ref[...]
    m1 = jnp.max(sv, axis=0)
    i1 = jnp.min(
        jnp.where(sv == m1[None], si, jnp.int32(0x7FFFFFFF)), axis=0
    )
    m = jnp.max(m1, axis=1, keepdims=True)
    isel = jnp.min(
        jnp.where(m1 == m, i1, jnp.int32(0x7FFFFFFF)), axis=1, keepdims=True
    )
    sc_wins = m >= tv_ref[...]
    out_ref[...] = jnp.where(sc_wins, isel, ti_ref[...])


def kernel(x, W, b):
    B, D = x.shape
    A = W.shape[0]
    b2 = b.reshape(1, A)

    lg_sc = pl.pallas_call(
        _sc_logits_kernel,
        grid=(_NW // _AGRP,),
        in_specs=[
            pl.BlockSpec((B, D), lambda t: (0, 0)),
            pl.BlockSpec((_AGRP * _P, D), lambda t: (t, 0)),
            pl.BlockSpec((1, _AGRP * _P), lambda t: (0, t)),
        ],
        out_specs=pl.BlockSpec((_AGRP, B, _P), lambda t: (t, 0, 0)),
        out_shape=jax.ShapeDtypeStruct((_NW, B, _P), jnp.float32),
    )(x, W, b2)

    sc_call = functools.partial(
        pl.kernel,
        mesh=plsc.VectorSubcoreMesh(core_axis_name="c", subcore_axis_name="s"),
        out_type=[
            jax.ShapeDtypeStruct((_NW, B, 16), jnp.float32),
            jax.ShapeDtypeStruct((_NW, B, 16), jnp.int32),
        ],
        scratch_types=[
            pltpu.VMEM((B, _P), jnp.float32),
            pltpu.VMEM((B, 16), jnp.float32),
            pltpu.VMEM((B, 16), jnp.int32),
        ],
    )(_sc_body)
    sc_val, sc_idx = sc_call(lg_sc)

    G = pl.cdiv(A - _A_SC, _TILE)
    tc_val, tc_idx = pl.pallas_call(
        functools.partial(_policy_kernel, num_blocks=G),
        grid=(G,),
        in_specs=[
            pl.BlockSpec((B, D), lambda i: (0, 0)),
            pl.BlockSpec((_TILE, D), lambda i: (_A_SC // _TILE + i, 0)),
            pl.BlockSpec((1, _TILE), lambda i: (0, _A_SC // _TILE + i)),
        ],
        out_specs=[
            pl.BlockSpec((B, 1), lambda i: (0, 0)),
            pl.BlockSpec((B, 1), lambda i: (0, 0)),
        ],
        out_shape=[
            jax.ShapeDtypeStruct((B, 1), jnp.float32),
            jax.ShapeDtypeStruct((B, 1), jnp.int32),
        ],
        scratch_shapes=[
            pltpu.VMEM((B, 1), jnp.float32),
            pltpu.VMEM((B, 1), jnp.int32),
            pltpu.VMEM((B, _TILE), jnp.int32),
        ],
    )(x, W, b2)

    sample = pl.pallas_call(
        _merge_kernel,
        in_specs=[
            pl.BlockSpec((_NW, B, 16), lambda: (0, 0, 0)),
            pl.BlockSpec((_NW, B, 16), lambda: (0, 0, 0)),
            pl.BlockSpec((B, 1), lambda: (0, 0)),
            pl.BlockSpec((B, 1), lambda: (0, 0)),
        ],
        out_specs=pl.BlockSpec((B, 1), lambda: (0, 0)),
        out_shape=jax.ShapeDtypeStruct((B, 1), jnp.int32),
    )(sc_val, sc_idx, tc_val, tc_idx)

    return sample.astype(jnp.int64)
